# Initial kernel scaffold; baseline (speedup 1.0000x reference)
import jax
import jax.numpy as jnp
from jax.experimental import pallas as pl


def kernel(x, kohonen_weights, grossberg_w, grossberg_b):
    raise NotImplementedError("write your pallas kernel here")


# speedup vs baseline: 6.1274x; 6.1274x over previous
import jax
import jax.numpy as jnp
from jax import lax
from jax.experimental import pallas as pl

_BM = 256


def _cpn_body(x_ref, kw_ref, gw_ref, gb_ref, win_ref, out_ref):
    K, D = kw_ref.shape
    xb = x_ref[...]
    kw = kw_ref[...]
    nrm = jnp.sqrt(jnp.sum(xb * xb, axis=1, keepdims=True))
    xn = xb / jnp.maximum(nrm, 1e-12)
    xsq = jnp.sum(xn * xn, axis=1, keepdims=True)
    xsq_t = jnp.transpose(xsq)
    wsq = jnp.sum(kw * kw, axis=1, keepdims=True)
    scores_t = lax.dot_general(
        kw, xn, (((1,), (1,)), ((), ())),
        preferred_element_type=jnp.float32)
    d2 = (xsq_t + wsq) - 2.0 * scores_t
    dist = jnp.sqrt(jnp.maximum(d2, 0.0))
    minv = jnp.min(dist, axis=0, keepdims=True)
    rows = lax.broadcasted_iota(jnp.int32, (K, _BM), 0)
    winners = jnp.min(jnp.where(dist == minv, rows, K),
                      axis=0, keepdims=True)
    win_ref[...] = winners[None]
    gwv = jnp.sum(jnp.where(rows == winners, gw_ref[...], 0.0),
                  axis=0, keepdims=True)
    out_ref[...] = jax.nn.sigmoid(gwv + gb_ref[0, 0])[None]


def kernel(x, kohonen_weights, grossberg_w, grossberg_b):
    B, D = x.shape
    K = kohonen_weights.shape[0]
    G = B // _BM
    gw_col = grossberg_w.reshape(K, 1)
    gb = grossberg_b.reshape(1, 1)
    win, out = pl.pallas_call(
        _cpn_body,
        grid=(G,),
        in_specs=[
            pl.BlockSpec((_BM, D), lambda i: (i, 0)),
            pl.BlockSpec((K, D), lambda i: (0, 0)),
            pl.BlockSpec((K, 1), lambda i: (0, 0)),
            pl.BlockSpec((1, 1), lambda i: (0, 0)),
        ],
        out_specs=[
            pl.BlockSpec((1, 1, _BM), lambda i: (i, 0, 0)),
            pl.BlockSpec((1, 1, _BM), lambda i: (i, 0, 0)),
        ],
        out_shape=[
            jax.ShapeDtypeStruct((G, 1, _BM), jnp.int32),
            jax.ShapeDtypeStruct((G, 1, _BM), jnp.float32),
        ],
    )(x, kohonen_weights, gw_col, gb)
    return out.reshape(B, 1), win.reshape(B)


# speedup vs baseline: 9.1775x; 1.4978x over previous
import jax
import jax.numpy as jnp
from jax import lax
from jax.experimental import pallas as pl

_BM = 256


def _cpn_body(x_ref, kw_ref, gw_ref, gb_ref, win_ref, out_ref):
    K, D = kw_ref.shape
    xb = x_ref[...]
    kw = kw_ref[...]
    nrm = jnp.sqrt(jnp.sum(xb * xb, axis=1, keepdims=True))
    xn = xb / jnp.maximum(nrm, 1e-12)
    xsq = jnp.sum(xn * xn, axis=1, keepdims=True)
    xsq_t = jnp.transpose(xsq)
    wsq = jnp.sum(kw * kw, axis=1, keepdims=True)
    scores_t = lax.dot_general(
        kw, xn, (((1,), (1,)), ((), ())),
        preferred_element_type=jnp.float32)
    d2 = (xsq_t + wsq) - 2.0 * scores_t
    minv = jnp.min(d2, axis=0, keepdims=True)
    rows = lax.broadcasted_iota(jnp.int32, (K, _BM), 0)
    mask = d2 == minv
    winners = jnp.min(jnp.where(mask, rows, K),
                      axis=0, keepdims=True)
    win_ref[...] = winners[None]
    gwv = jnp.min(jnp.where(mask, gw_ref[...], jnp.inf),
                  axis=0, keepdims=True)
    out_ref[...] = jax.nn.sigmoid(gwv + gb_ref[0, 0])[None]


def kernel(x, kohonen_weights, grossberg_w, grossberg_b):
    B, D = x.shape
    K = kohonen_weights.shape[0]
    G = B // _BM
    gw_col = grossberg_w.reshape(K, 1)
    gb = grossberg_b.reshape(1, 1)
    win, out = pl.pallas_call(
        _cpn_body,
        grid=(G,),
        in_specs=[
            pl.BlockSpec((_BM, D), lambda i: (i, 0)),
            pl.BlockSpec((K, D), lambda i: (0, 0)),
            pl.BlockSpec((K, 1), lambda i: (0, 0)),
            pl.BlockSpec((1, 1), lambda i: (0, 0)),
        ],
        out_specs=[
            pl.BlockSpec((1, 1, _BM), lambda i: (i, 0, 0)),
            pl.BlockSpec((1, 1, _BM), lambda i: (i, 0, 0)),
        ],
        out_shape=[
            jax.ShapeDtypeStruct((G, 1, _BM), jnp.int32),
            jax.ShapeDtypeStruct((G, 1, _BM), jnp.float32),
        ],
    )(x, kohonen_weights, gw_col, gb)
    return out.reshape(B, 1), win.reshape(B)
